# baseline (speedup 1.0000x reference)
import jax
import jax.numpy as jnp
from jax import lax
from jax.experimental import pallas as pl
from jax.experimental.pallas import tpu as pltpu

_ZCH = 1_000_000
_B = 31232
_NSUB = 4
_B8 = _B // _NSUB
_NCH = 32
_TAILC = _NCH * _B
_TAIL = _ZCH - _TAILC


def _body(x_hbm, cnt_ref, ratio_ref, buf, tbuf, *sems_all):
    sems = list(sems_all[:-1])
    ts = sems_all[-1]

    def cps(k):
        base = k * _B
        return [
            pltpu.make_async_copy(
                x_hbm.at[0, pl.ds(base + j * _B8, _B8)],
                buf.at[k, j],
                sems[k],
            )
            for j in range(_NSUB)
        ]

    for k in range(_NCH):
        for c in cps(k):
            c.start()
    tc = pltpu.make_async_copy(x_hbm.at[0, pl.ds(_TAILC, _TAIL)], tbuf, ts)
    tc.start()

    acc = jnp.zeros((_NSUB, 128), jnp.int32)
    ones = jnp.ones((_NSUB, _B8), jnp.int32)
    zeros = jnp.zeros((_NSUB, _B8), jnp.int32)
    for k in range(_NCH):
        for c in cps(k):
            c.wait()
        hit = jnp.where(buf[k] == -1, ones, zeros)
        acc = acc + jnp.sum(hit.reshape(_NSUB, _B8 // 128, 128), axis=1)

    total = jnp.sum(acc)
    tc.wait()
    total = total + jnp.sum((tbuf[...] == -1).astype(jnp.int32))

    cnt_ref[0, 0] = total
    ratio_ref[0, 0] = (
        jnp.float32(_ZCH) - total.astype(jnp.float32)
    ) / jnp.float32(_ZCH)


def kernel(identities):
    idT = identities.T
    cnt, ratio = pl.pallas_call(
        _body,
        compiler_params=pltpu.CompilerParams(
            vmem_limit_bytes=56 * 1024 * 1024
        ),
        in_specs=[pl.BlockSpec(memory_space=pl.ANY)],
        out_specs=(
            pl.BlockSpec(memory_space=pltpu.SMEM),
            pl.BlockSpec(memory_space=pltpu.SMEM),
        ),
        out_shape=(
            jax.ShapeDtypeStruct((1, 1), jnp.int32),
            jax.ShapeDtypeStruct((1, 1), jnp.float32),
        ),
        scratch_shapes=[
            pltpu.VMEM((_NCH, _NSUB, _B8), jnp.int32),
            pltpu.VMEM((_TAIL,), jnp.int32),
        ] + [pltpu.SemaphoreType.DMA] * (_NCH + 1),
    )(idT)
    return cnt[0, 0], ratio[0, 0]


# speedup vs baseline: 1.0474x; 1.0474x over previous
import jax
import jax.numpy as jnp
from jax import lax
from jax.experimental import pallas as pl
from jax.experimental.pallas import tpu as pltpu

_ZCH = 1_000_000
_B = 62464
_NSUB = 4
_B8 = _B // _NSUB
_NCH = 16
_TAILC = _NCH * _B
_TAIL = _ZCH - _TAILC


def _body(x_hbm, cnt_ref, ratio_ref, buf, tbuf, *sems_all):
    sems = list(sems_all[:-1])
    ts = sems_all[-1]

    def cps(k):
        base = k * _B
        return [
            pltpu.make_async_copy(
                x_hbm.at[0, pl.ds(base + j * _B8, _B8)],
                buf.at[k, j],
                sems[k],
            )
            for j in range(_NSUB)
        ]

    for k in range(_NCH):
        for c in cps(k):
            c.start()
    tc = pltpu.make_async_copy(x_hbm.at[0, pl.ds(_TAILC, _TAIL)], tbuf, ts)
    tc.start()

    acc = jnp.zeros((_NSUB, 128), jnp.int32)
    ones = jnp.ones((_NSUB, _B8), jnp.int32)
    zeros = jnp.zeros((_NSUB, _B8), jnp.int32)
    for k in range(_NCH):
        for c in cps(k):
            c.wait()
        hit = jnp.where(buf[k] == -1, ones, zeros)
        acc = acc + jnp.sum(hit.reshape(_NSUB, _B8 // 128, 128), axis=1)

    total = jnp.sum(acc)
    tc.wait()
    total = total + jnp.sum((tbuf[...] == -1).astype(jnp.int32))

    cnt_ref[0, 0] = total
    ratio_ref[0, 0] = (
        jnp.float32(_ZCH) - total.astype(jnp.float32)
    ) / jnp.float32(_ZCH)


def kernel(identities):
    idT = identities.T
    cnt, ratio = pl.pallas_call(
        _body,
        compiler_params=pltpu.CompilerParams(
            vmem_limit_bytes=56 * 1024 * 1024
        ),
        in_specs=[pl.BlockSpec(memory_space=pl.ANY)],
        out_specs=(
            pl.BlockSpec(memory_space=pltpu.SMEM),
            pl.BlockSpec(memory_space=pltpu.SMEM),
        ),
        out_shape=(
            jax.ShapeDtypeStruct((1, 1), jnp.int32),
            jax.ShapeDtypeStruct((1, 1), jnp.float32),
        ),
        scratch_shapes=[
            pltpu.VMEM((_NCH, _NSUB, _B8), jnp.int32),
            pltpu.VMEM((_TAIL,), jnp.int32),
        ] + [pltpu.SemaphoreType.DMA] * (_NCH + 1),
    )(idT)
    return cnt[0, 0], ratio[0, 0]


# speedup vs baseline: 1.0609x; 1.0130x over previous
import jax
import jax.numpy as jnp
from jax import lax
from jax.experimental import pallas as pl
from jax.experimental.pallas import tpu as pltpu

_ZCH = 1_000_000
_B = 62464
_NSUB = 8
_B8 = _B // _NSUB
_NCH = 16
_TAILC = _NCH * _B
_TAIL = _ZCH - _TAILC


def _body(x_hbm, cnt_ref, ratio_ref, buf, tbuf, *sems_all):
    sems = list(sems_all[:-1])
    ts = sems_all[-1]

    def cps(k):
        base = k * _B
        return [
            pltpu.make_async_copy(
                x_hbm.at[0, pl.ds(base + j * _B8, _B8)],
                buf.at[k, j],
                sems[k],
            )
            for j in range(_NSUB)
        ]

    for k in range(_NCH):
        for c in cps(k):
            c.start()
    tc = pltpu.make_async_copy(x_hbm.at[0, pl.ds(_TAILC, _TAIL)], tbuf, ts)
    tc.start()

    acc = jnp.zeros((_NSUB, 128), jnp.int32)
    ones = jnp.ones((_NSUB, _B8), jnp.int32)
    zeros = jnp.zeros((_NSUB, _B8), jnp.int32)
    for k in range(_NCH):
        for c in cps(k):
            c.wait()
        hit = jnp.where(buf[k] == -1, ones, zeros)
        acc = acc + jnp.sum(hit.reshape(_NSUB, _B8 // 128, 128), axis=1)

    total = jnp.sum(acc)
    tc.wait()
    total = total + jnp.sum((tbuf[...] == -1).astype(jnp.int32))

    cnt_ref[0, 0] = total
    ratio_ref[0, 0] = (
        jnp.float32(_ZCH) - total.astype(jnp.float32)
    ) / jnp.float32(_ZCH)


def kernel(identities):
    idT = identities.T
    cnt, ratio = pl.pallas_call(
        _body,
        compiler_params=pltpu.CompilerParams(
            vmem_limit_bytes=56 * 1024 * 1024
        ),
        in_specs=[pl.BlockSpec(memory_space=pl.ANY)],
        out_specs=(
            pl.BlockSpec(memory_space=pltpu.SMEM),
            pl.BlockSpec(memory_space=pltpu.SMEM),
        ),
        out_shape=(
            jax.ShapeDtypeStruct((1, 1), jnp.int32),
            jax.ShapeDtypeStruct((1, 1), jnp.float32),
        ),
        scratch_shapes=[
            pltpu.VMEM((_NCH, _NSUB, _B8), jnp.int32),
            pltpu.VMEM((_TAIL,), jnp.int32),
        ] + [pltpu.SemaphoreType.DMA] * (_NCH + 1),
    )(idT)
    return cnt[0, 0], ratio[0, 0]


# speedup vs baseline: 1.0656x; 1.0044x over previous
import jax
import jax.numpy as jnp
from jax.experimental import pallas as pl
from jax.experimental.pallas import tpu as pltpu

_ZCH = 1_000_000
_B = 62464
_NSUB = 8
_B8 = _B // _NSUB
_NCH = 16
_TAILC = _NCH * _B
_TAIL = _ZCH - _TAILC


def _body(x_hbm, cnt_ref, ratio_ref, buf, tbuf, *sems_all):
    sems = list(sems_all[:-1])
    ts = sems_all[-1]

    def cps(k):
        base = k * _B
        return [
            pltpu.make_async_copy(
                x_hbm.at[0, pl.ds(base + j * _B8, _B8)],
                buf.at[k, j],
                sems[k],
            )
            for j in range(_NSUB)
        ]

    for k in range(_NCH):
        for c in cps(k):
            c.start()
    tc = pltpu.make_async_copy(x_hbm.at[0, pl.ds(_TAILC, _TAIL)], tbuf, ts)
    tc.start()

    acc = jnp.zeros((_NSUB, 128), jnp.int32)
    ones = jnp.ones((_NSUB, _B8), jnp.int32)
    zeros = jnp.zeros((_NSUB, _B8), jnp.int32)
    for k in range(_NCH):
        for c in cps(k):
            c.wait()
        hit = jnp.where(buf[k] == -1, ones, zeros)
        acc = acc + jnp.sum(hit.reshape(_NSUB, _B8 // 128, 128), axis=1)

    total = jnp.sum(acc)
    tc.wait()
    total = total + jnp.sum((tbuf[...] == -1).astype(jnp.int32))

    cnt_ref[0, 0] = total
    ratio_ref[0, 0] = (
        jnp.float32(_ZCH) - total.astype(jnp.float32)
    ) / jnp.float32(_ZCH)


def kernel(identities):
    idT = identities.T
    cnt, ratio = pl.pallas_call(
        _body,
        compiler_params=pltpu.CompilerParams(
            vmem_limit_bytes=56 * 1024 * 1024
        ),
        in_specs=[pl.BlockSpec(memory_space=pl.ANY)],
        out_specs=(
            pl.BlockSpec(memory_space=pltpu.SMEM),
            pl.BlockSpec(memory_space=pltpu.SMEM),
        ),
        out_shape=(
            jax.ShapeDtypeStruct((1, 1), jnp.int32),
            jax.ShapeDtypeStruct((1, 1), jnp.float32),
        ),
        scratch_shapes=[
            pltpu.VMEM((_NCH, _NSUB, _B8), jnp.int32),
            pltpu.VMEM((_TAIL,), jnp.int32),
        ] + [pltpu.SemaphoreType.DMA] * (_NCH + 1),
    )(idT)
    return cnt[0, 0], ratio[0, 0]
